# Initial kernel scaffold; baseline (speedup 1.0000x reference)
import jax
import jax.numpy as jnp
from jax.experimental import pallas as pl


def kernel(x, edge_index, W1a, b1a, W1b, b1b, W1c, b1c, W2a, b2a, W2b, b2b, W2c, b2c, Wo, bo):
    raise NotImplementedError("write your pallas kernel here")


# speedup vs baseline: 3.5843x; 3.5843x over previous
import functools

import jax
import jax.numpy as jnp
from jax import lax
from jax.experimental import pallas as pl
from jax.experimental.pallas import tpu as pltpu
from jax.experimental.pallas import tpu_sc as plsc

N = 10000
E = 320000
NP = 10240
CHUNK = 128
NW = 32
EW = (E + NW * CHUNK - 1) // (NW * CHUNK)
EPAD = NW * EW * CHUNK
STRIPE = NP // 16


D = 128


@functools.lru_cache(maxsize=None)
def _make_agg(with_deg):
    mesh = plsc.VectorSubcoreMesh(
        core_axis_name="c", subcore_axis_name="s", num_cores=2, num_subcores=16)

    out_type = [jax.ShapeDtypeStruct((2, NP, D), jnp.float32)]
    scratch = [
        pltpu.VMEM((EW, CHUNK), jnp.int32),
        pltpu.VMEM((EW, CHUNK), jnp.int32),
        pltpu.VMEM((CHUNK, D), jnp.float32),
        pltpu.VMEM_SHARED((NP, D), jnp.float32),
        pltpu.SemaphoreType.DMA,
    ]
    if with_deg:
        out_type.append(jax.ShapeDtypeStruct((NW, NP), jnp.float32))
        scratch.append(pltpu.VMEM((NP,), jnp.float32))

    def body(table, srcp, dstp, out, deg_out, src_v, dst_v, buf, acc,
             sem, degtab):
        c = lax.axis_index("c")
        s = lax.axis_index("s")
        wid = c * 16 + s

        zero = jnp.zeros((16,), jnp.float32)

        def zrow(i, carry):
            for k in range(D // 16):
                buf[i, k * 16:(k + 1) * 16] = zero
            return carry

        lax.fori_loop(0, CHUNK, zrow, 0)
        for t in range(STRIPE // CHUNK):
            pltpu.sync_copy(buf, acc.at[pl.ds(s * STRIPE + t * CHUNK, CHUNK)])

        pltpu.sync_copy(srcp.at[wid], src_v)
        pltpu.sync_copy(dstp.at[wid], dst_v)

        if with_deg:
            def dzero(i, carry):
                degtab[pl.ds(i * 16, 16)] = zero
                return carry

            lax.fori_loop(0, NP // 16, dzero, 0)
            ones16 = jnp.ones((16,), jnp.float32)

            def dbody(j, carry):
                for k in range(CHUNK // 16):
                    idx = dst_v[j, k * 16:(k + 1) * 16]
                    plsc.addupdate_scatter(degtab, [idx], ones16)
                return carry

            lax.fori_loop(0, EW, dbody, 0)
            pltpu.sync_copy(degtab, deg_out.at[wid])

        plsc.subcore_barrier()

        def ebody(j, carry):
            pltpu.async_copy(table.at[src_v.at[j]], buf, sem).wait()
            pltpu.sync_copy(buf, acc.at[dst_v.at[j]], add=True)
            return carry

        lax.fori_loop(0, EW, ebody, 0)
        plsc.subcore_barrier()

        for t in range(STRIPE // CHUNK):
            sl = pl.ds(s * STRIPE + t * CHUNK, CHUNK)
            pltpu.sync_copy(acc.at[sl], out.at[c, sl])

    if with_deg:
        def agg_body(table, srcp, dstp, out, deg_out, src_v, dst_v, buf,
                     acc, sem, degtab):
            body(table, srcp, dstp, out, deg_out, src_v, dst_v, buf,
                 acc, sem, degtab)
    else:
        def agg_body(table, srcp, dstp, out, src_v, dst_v, buf, acc, sem):
            body(table, srcp, dstp, out, None, src_v, dst_v, buf, acc,
                 sem, None)

    if not with_deg:
        out_type = out_type[0]
    return functools.partial(
        pl.kernel, mesh=mesh, out_type=out_type, scratch_types=scratch,
        compiler_params=pltpu.CompilerParams(needs_layout_passes=False),
    )(agg_body)


def _mlp1_body(x_ref, agg_ref, degp_ref, wa, ba, wb, bb, wc, bc,
               ha_ref, hb_ref, inv_ref):
    deg = lax.dot_general(degp_ref[...], jnp.ones((NW, 1), jnp.float32),
                          (((0,), (0,)), ((), ())),
                          preferred_element_type=jnp.float32)
    inv = 1.0 / jnp.maximum(deg, 1.0)
    agg = agg_ref[0] + agg_ref[1]
    z = x_ref[...] + agg * inv
    z = jax.nn.relu(jnp.dot(z, wa[...], preferred_element_type=jnp.float32) + ba[...])
    z = jax.nn.relu(jnp.dot(z, wb[...], preferred_element_type=jnp.float32) + bb[...])
    h = jax.nn.relu(jnp.dot(z, wc[...], preferred_element_type=jnp.float32) + bc[...])
    ha_ref[...] = h[:, :128]
    hb_ref[...] = h[:, 128:]
    inv_ref[...] = jnp.broadcast_to(inv, (128, 8))


def _mlp1(x_pad, agg1, degp, wa, ba, wb, bb, wc, bc):
    full = lambda shape: pl.BlockSpec(shape, lambda i: (0,) * len(shape))
    return pl.pallas_call(
        _mlp1_body,
        grid=(NP // 128,),
        in_specs=[
            pl.BlockSpec((128, 128), lambda i: (i, 0)),
            pl.BlockSpec((2, 128, 128), lambda i: (0, i, 0)),
            pl.BlockSpec((NW, 128), lambda i: (0, i)),
            full((128, 128)), full((1, 128)),
            full((128, 256)), full((1, 256)),
            full((256, 256)), full((1, 256)),
        ],
        out_specs=[
            pl.BlockSpec((128, 128), lambda i: (i, 0)),
            pl.BlockSpec((128, 128), lambda i: (i, 0)),
            pl.BlockSpec((128, 8), lambda i: (i, 0)),
        ],
        out_shape=[
            jax.ShapeDtypeStruct((NP, 128), jnp.float32),
            jax.ShapeDtypeStruct((NP, 128), jnp.float32),
            jax.ShapeDtypeStruct((NP, 8), jnp.float32),
        ],
    )(x_pad, agg1, degp, wa, ba, wb, bb, wc, bc)


def _mlp2_body(ha_ref, hb_ref, aggA_ref, aggB_ref, inv_ref,
               wa, ba, wb, bb, wc, bc, wo, bo, out_ref):
    inv = inv_ref[...][:, 0:1]
    ma = (aggA_ref[0] + aggA_ref[1]) * inv
    mb = (aggB_ref[0] + aggB_ref[1]) * inv
    z = jnp.concatenate([ha_ref[...] + ma, hb_ref[...] + mb], axis=1)
    z = jax.nn.relu(jnp.dot(z, wa[...], preferred_element_type=jnp.float32) + ba[...])
    z = jax.nn.relu(jnp.dot(z, wb[...], preferred_element_type=jnp.float32) + bb[...])
    z = jnp.dot(z, wc[...], preferred_element_type=jnp.float32) + bc[...]
    h2 = jax.nn.relu(z)
    out_ref[...] = jnp.dot(h2, wo[...], preferred_element_type=jnp.float32) + bo[...]


def _mlp2(ha, hb, aggA, aggB, invd, wa, ba, wb, bb, wc, bc, wo, bo):
    full = lambda shape: pl.BlockSpec(shape, lambda i: (0,) * len(shape))
    return pl.pallas_call(
        _mlp2_body,
        grid=(NP // 128,),
        in_specs=[
            pl.BlockSpec((128, 128), lambda i: (i, 0)),
            pl.BlockSpec((128, 128), lambda i: (i, 0)),
            pl.BlockSpec((2, 128, 128), lambda i: (0, i, 0)),
            pl.BlockSpec((2, 128, 128), lambda i: (0, i, 0)),
            pl.BlockSpec((128, 8), lambda i: (i, 0)),
            full((256, 256)), full((1, 256)),
            full((256, 256)), full((1, 256)),
            full((256, 256)), full((1, 256)),
            full((256, 64)), full((1, 64)),
        ],
        out_specs=pl.BlockSpec((128, 64), lambda i: (i, 0)),
        out_shape=jax.ShapeDtypeStruct((NP, 64), jnp.float32),
    )(ha, hb, aggA, aggB, invd, wa, ba, wb, bb, wc, bc, wo, bo)


def kernel(x, edge_index, W1a, b1a, W1b, b1b, W1c, b1c,
           W2a, b2a, W2b, b2b, W2c, b2c, Wo, bo):
    src = edge_index[0]
    dst = edge_index[1]
    pad = jnp.full((EPAD - E,), N, jnp.int32)
    srcp = jnp.concatenate([src, pad]).reshape(NW, EW, CHUNK)
    dstp = jnp.concatenate([dst, pad]).reshape(NW, EW, CHUNK)

    x_pad = jnp.pad(x, ((0, NP - N), (0, 0)))

    agg1, degp = _make_agg(True)(x_pad, srcp, dstp)
    ha, hb, invd = _mlp1(
        x_pad, agg1, degp,
        W1a, b1a.reshape(1, -1), W1b, b1b.reshape(1, -1), W1c, b1c.reshape(1, -1))

    aggA = _make_agg(False)(ha, srcp, dstp)
    aggB = _make_agg(False)(hb, srcp, dstp)
    out = _mlp2(
        ha, hb, aggA, aggB, invd,
        W2a, b2a.reshape(1, -1), W2b, b2b.reshape(1, -1),
        W2c, b2c.reshape(1, -1), Wo, bo.reshape(1, -1))
    return out[:N]


# speedup vs baseline: 6.3276x; 1.7654x over previous
import functools

import jax
import jax.numpy as jnp
from jax import lax
from jax.experimental import pallas as pl
from jax.experimental.pallas import tpu as pltpu
from jax.experimental.pallas import tpu_sc as plsc

N = 10000
E = 320000
NP = 10240
CHUNK = 80
NW = 32
EW = (E + NW * CHUNK - 1) // (NW * CHUNK)
EPAD = NW * EW * CHUNK
STRIPE = NP // 16


D = 128


@functools.lru_cache(maxsize=None)
def _make_agg(with_deg):
    mesh = plsc.VectorSubcoreMesh(
        core_axis_name="c", subcore_axis_name="s", num_cores=2, num_subcores=16)

    out_type = [jax.ShapeDtypeStruct((2, NP, D), jnp.float32)]
    scratch = [
        pltpu.VMEM((EW, CHUNK), jnp.int32),
        pltpu.VMEM((CHUNK, D), jnp.float32),
        pltpu.VMEM((CHUNK, D), jnp.float32),
        pltpu.VMEM((CHUNK,), jnp.int32),
        pltpu.VMEM((CHUNK,), jnp.int32),
        pltpu.VMEM((CHUNK,), jnp.int32),
        pltpu.VMEM((CHUNK,), jnp.int32),
        pltpu.VMEM_SHARED((NP, D), jnp.float32),
        pltpu.SemaphoreType.DMA,
        pltpu.SemaphoreType.DMA,
    ]
    if with_deg:
        out_type.append(jax.ShapeDtypeStruct((NW, NP), jnp.float32))
        scratch.append(pltpu.VMEM((NP,), jnp.float32))

    def body(table, packed, out, deg_out, idx_v, buf0, buf1, src0, src1,
             dst0, dst1, acc, gsem0, gsem1, degtab):
        c = lax.axis_index("c")
        s = lax.axis_index("s")
        wid = c * 16 + s

        zero = jnp.zeros((16,), jnp.float32)

        def zrow(i, carry):
            for k in range(D // 16):
                buf0[i, k * 16:(k + 1) * 16] = zero
            return carry

        lax.fori_loop(0, CHUNK, zrow, 0)
        for t in range(STRIPE // CHUNK):
            pltpu.sync_copy(buf0, acc.at[pl.ds(s * STRIPE + t * CHUNK, CHUNK)])

        pltpu.sync_copy(packed.at[wid], idx_v)

        if with_deg:
            def dzero(i, carry):
                degtab[pl.ds(i * 16, 16)] = zero
                return carry

            lax.fori_loop(0, NP // 16, dzero, 0)
        plsc.subcore_barrier()

        ones16 = jnp.ones((16,), jnp.float32)
        mask16 = jnp.int32(0xFFFF)

        def prep(j, srow, drow):
            for k in range(CHUNK // 16):
                v = idx_v[j, k * 16:(k + 1) * 16]
                dvec = lax.shift_right_logical(v, 16)
                srow[k * 16:(k + 1) * 16] = v & mask16
                drow[k * 16:(k + 1) * 16] = dvec
                if with_deg:
                    plsc.addupdate_scatter(degtab, [dvec], ones16)

        def gather(buf, srow, gsem):
            pltpu.async_copy(table.at[srow], buf, gsem)

        def gwait(buf, gsem):
            pltpu.make_async_copy(table.at[src0], buf, gsem).wait()

        def scatter(buf, drow):
            pltpu.sync_copy(buf, acc.at[drow], add=True)

        prep(0, src0, dst0)
        gather(buf0, src0, gsem0)

        def pair(jj, carry):
            j0 = 2 * jj
            j1 = j0 + 1
            prep(j1, src1, dst1)
            gwait(buf0, gsem0)
            gather(buf1, src1, gsem1)
            scatter(buf0, dst0)
            prep(j0 + 2, src0, dst0)
            gwait(buf1, gsem1)
            gather(buf0, src0, gsem0)
            scatter(buf1, dst1)
            return carry

        lax.fori_loop(0, EW // 2 - 1, pair, 0)
        prep(EW - 1, src1, dst1)
        gwait(buf0, gsem0)
        gather(buf1, src1, gsem1)
        scatter(buf0, dst0)
        gwait(buf1, gsem1)
        scatter(buf1, dst1)

        if with_deg:
            pltpu.sync_copy(degtab, deg_out.at[wid])
        plsc.subcore_barrier()

        for t in range(STRIPE // CHUNK):
            sl = pl.ds(s * STRIPE + t * CHUNK, CHUNK)
            pltpu.sync_copy(acc.at[sl], out.at[c, sl])

    if with_deg:
        def agg_body(table, packed, out, deg_out, idx_v, buf0, buf1, src0,
                     src1, dst0, dst1, acc, gsem0, gsem1, degtab):
            body(table, packed, out, deg_out, idx_v, buf0, buf1, src0, src1,
                 dst0, dst1, acc, gsem0, gsem1, degtab)
    else:
        def agg_body(table, packed, out, idx_v, buf0, buf1, src0, src1,
                     dst0, dst1, acc, gsem0, gsem1):
            body(table, packed, out, None, idx_v, buf0, buf1, src0, src1,
                 dst0, dst1, acc, gsem0, gsem1, None)

    if not with_deg:
        out_type = out_type[0]
    return functools.partial(
        pl.kernel, mesh=mesh, out_type=out_type, scratch_types=scratch,
        compiler_params=pltpu.CompilerParams(needs_layout_passes=False),
    )(agg_body)


def _mlp1_body(x_ref, agg_ref, degp_ref, wa, ba, wb, bb, wc, bc,
               ha_ref, hb_ref, inv_ref):
    deg = lax.dot_general(degp_ref[...], jnp.ones((NW, 1), jnp.float32),
                          (((0,), (0,)), ((), ())),
                          preferred_element_type=jnp.float32)
    inv = 1.0 / jnp.maximum(deg, 1.0)
    agg = agg_ref[0] + agg_ref[1]
    z = x_ref[...] + agg * inv
    z = jax.nn.relu(jnp.dot(z, wa[...], preferred_element_type=jnp.float32) + ba[...])
    z = jax.nn.relu(jnp.dot(z, wb[...], preferred_element_type=jnp.float32) + bb[...])
    h = jax.nn.relu(jnp.dot(z, wc[...], preferred_element_type=jnp.float32) + bc[...])
    ha_ref[...] = h[:, :128]
    hb_ref[...] = h[:, 128:]
    inv_ref[...] = jnp.broadcast_to(inv, (128, 8))


def _mlp1(x_pad, agg1, degp, wa, ba, wb, bb, wc, bc):
    full = lambda shape: pl.BlockSpec(shape, lambda i: (0,) * len(shape))
    return pl.pallas_call(
        _mlp1_body,
        grid=(NP // 128,),
        in_specs=[
            pl.BlockSpec((128, 128), lambda i: (i, 0)),
            pl.BlockSpec((2, 128, 128), lambda i: (0, i, 0)),
            pl.BlockSpec((NW, 128), lambda i: (0, i)),
            full((128, 128)), full((1, 128)),
            full((128, 256)), full((1, 256)),
            full((256, 256)), full((1, 256)),
        ],
        out_specs=[
            pl.BlockSpec((128, 128), lambda i: (i, 0)),
            pl.BlockSpec((128, 128), lambda i: (i, 0)),
            pl.BlockSpec((128, 8), lambda i: (i, 0)),
        ],
        out_shape=[
            jax.ShapeDtypeStruct((NP, 128), jnp.float32),
            jax.ShapeDtypeStruct((NP, 128), jnp.float32),
            jax.ShapeDtypeStruct((NP, 8), jnp.float32),
        ],
    )(x_pad, agg1, degp, wa, ba, wb, bb, wc, bc)


def _mlp2_body(ha_ref, hb_ref, aggA_ref, aggB_ref, inv_ref,
               wa, ba, wb, bb, wc, bc, wo, bo, out_ref):
    inv = inv_ref[...][:, 0:1]
    ma = (aggA_ref[0] + aggA_ref[1]) * inv
    mb = (aggB_ref[0] + aggB_ref[1]) * inv
    z = jnp.concatenate([ha_ref[...] + ma, hb_ref[...] + mb], axis=1)
    z = jax.nn.relu(jnp.dot(z, wa[...], preferred_element_type=jnp.float32) + ba[...])
    z = jax.nn.relu(jnp.dot(z, wb[...], preferred_element_type=jnp.float32) + bb[...])
    z = jnp.dot(z, wc[...], preferred_element_type=jnp.float32) + bc[...]
    h2 = jax.nn.relu(z)
    out_ref[...] = jnp.dot(h2, wo[...], preferred_element_type=jnp.float32) + bo[...]


def _mlp2(ha, hb, aggA, aggB, invd, wa, ba, wb, bb, wc, bc, wo, bo):
    full = lambda shape: pl.BlockSpec(shape, lambda i: (0,) * len(shape))
    return pl.pallas_call(
        _mlp2_body,
        grid=(NP // 128,),
        in_specs=[
            pl.BlockSpec((128, 128), lambda i: (i, 0)),
            pl.BlockSpec((128, 128), lambda i: (i, 0)),
            pl.BlockSpec((2, 128, 128), lambda i: (0, i, 0)),
            pl.BlockSpec((2, 128, 128), lambda i: (0, i, 0)),
            pl.BlockSpec((128, 8), lambda i: (i, 0)),
            full((256, 256)), full((1, 256)),
            full((256, 256)), full((1, 256)),
            full((256, 256)), full((1, 256)),
            full((256, 64)), full((1, 64)),
        ],
        out_specs=pl.BlockSpec((128, 64), lambda i: (i, 0)),
        out_shape=jax.ShapeDtypeStruct((NP, 64), jnp.float32),
    )(ha, hb, aggA, aggB, invd, wa, ba, wb, bb, wc, bc, wo, bo)


def kernel(x, edge_index, W1a, b1a, W1b, b1b, W1c, b1c,
           W2a, b2a, W2b, b2b, W2c, b2c, Wo, bo):
    src = edge_index[0]
    dst = edge_index[1]
    pad = jnp.full((EPAD - E,), N, jnp.int32)
    packed = (jnp.concatenate([src, pad])
              | (jnp.concatenate([dst, pad]) << 16)).reshape(NW, EW, CHUNK)

    x_pad = jnp.pad(x, ((0, NP - N), (0, 0)))

    agg1, degp = _make_agg(True)(x_pad, packed)
    ha, hb, invd = _mlp1(
        x_pad, agg1, degp,
        W1a, b1a.reshape(1, -1), W1b, b1b.reshape(1, -1), W1c, b1c.reshape(1, -1))

    aggA = _make_agg(False)(ha, packed)
    aggB = _make_agg(False)(hb, packed)
    out = _mlp2(
        ha, hb, aggA, aggB, invd,
        W2a, b2a.reshape(1, -1), W2b, b2b.reshape(1, -1),
        W2c, b2c.reshape(1, -1), Wo, bo.reshape(1, -1))
    return out[:N]
